# baseline (speedup 1.0000x reference)
import functools

import jax
import jax.numpy as jnp
from jax import lax
from jax.experimental import pallas as pl
from jax.experimental.pallas import tpu as pltpu
from jax.experimental.pallas import tpu_sc as plsc

_SCALE = 8.0

_RING = 4


def _scale_pad_body(w_ref, o_ref):
    w = w_ref[...]
    o_ref[...] = jnp.concatenate([w * _SCALE, jnp.zeros_like(w)], axis=1)


def _scale_pad_table(w):
    v, d = w.shape
    blk = 10000
    assert v % blk == 0 and blk % 8 == 0
    return pl.pallas_call(
        _scale_pad_body,
        out_shape=jax.ShapeDtypeStruct((v, 2 * d), w.dtype),
        grid=(v // blk,),
        in_specs=[pl.BlockSpec((blk, d), lambda i: (i, 0))],
        out_specs=pl.BlockSpec((blk, 2 * d), lambda i: (i, 0)),
    )(w)


@functools.partial(jax.jit, static_argnums=(2, 3, 4))
def _sc_gather_t(idx_t, table, b, l, d):
    info = plsc.get_sparse_core_info()
    nl = info.num_lanes
    nw = info.num_cores * info.num_subcores
    pbw = b // nw
    ng = l // _RING
    assert pbw * nw == b and ng * _RING == l and pbw % nl == 0

    mesh = plsc.VectorSubcoreMesh(core_axis_name="c", subcore_axis_name="s")

    @functools.partial(
        pl.kernel,
        mesh=mesh,
        compiler_params=pltpu.CompilerParams(needs_layout_passes=False),
        out_type=jax.ShapeDtypeStruct((l, d, b), jnp.float32),
        scratch_types=(
            [pltpu.VMEM((l, pbw), jnp.int32)]
            + [pltpu.VMEM((pbw, 2 * d), jnp.float32) for _ in range(_RING)]
            + [pltpu.VMEM((d, pbw), jnp.float32) for _ in range(2)]
            + [pltpu.SemaphoreType.DMA for _ in range(_RING + 2)]
        ),
    )
    def k(idxt_hbm, table_hbm, out_hbm, idx_v, *rest):
        gbufs = rest[:_RING]
        sbufs = rest[_RING:_RING + 2]
        sems = rest[_RING + 2:2 * _RING + 2]
        osems = rest[2 * _RING + 2:]
        wid = lax.axis_index("s") * info.num_cores + lax.axis_index("c")
        base = wid * pbw
        pltpu.sync_copy(idxt_hbm.at[:, pl.ds(base, pbw)], idx_v)

        def fire(i, r):
            pltpu.async_copy(table_hbm.at[idx_v.at[i]], gbufs[r], sems[r])

        def drain(i, r):
            pltpu.make_async_copy(
                table_hbm.at[idx_v.at[i]], gbufs[r], sems[r]).wait()

        def transpose(r, s):
            iota = lax.iota(jnp.int32, nl)
            perm = [(iota + k) & (nl - 1) for k in range(nl)]

            @plsc.parallel_loop(0, pbw // nl, unroll=2)
            def _(jb):
                rows = jb * nl + iota
                for e0 in range(0, d, nl):
                    for k0 in range(0, nl, 8):
                        vals = [
                            plsc.load_gather(gbufs[r], [rows, perm[k0 + k] + e0])
                            for k in range(8)
                        ]
                        for k in range(8):
                            plsc.store_scatter(
                                sbufs[s], [perm[k0 + k] + e0, rows], vals[k])

        for r in range(_RING):
            fire(r, r)

        def body(g, carry):
            for r in range(_RING):
                s = r % 2
                i = g * _RING + r
                drain(i, r)

                @pl.when(i >= 2)
                def _():
                    pltpu.make_async_copy(
                        sbufs[s], out_hbm.at[i - 2, :, pl.ds(base, pbw)],
                        osems[s]).wait()

                transpose(r, s)
                pltpu.async_copy(
                    sbufs[s], out_hbm.at[i, :, pl.ds(base, pbw)], osems[s])
                ni = i + _RING

                @pl.when(ni < l)
                def _():
                    fire(ni, r)
            return carry

        lax.fori_loop(0, ng, body, None)
        for s in range(2):
            pltpu.make_async_copy(
                sbufs[s], out_hbm.at[l - 2 + s, :, pl.ds(base, pbw)],
                osems[s]).wait()

    return k(idx_t, table)


def kernel(tokens, word_vectors):
    b, l = tokens.shape
    v, d = word_vectors.shape
    scaled = _scale_pad_table(word_vectors)
    out3 = _sc_gather_t(tokens.T, scaled, b, l, d)
    return jnp.transpose(out3, (2, 0, 1))


# speedup vs baseline: 2.1592x; 2.1592x over previous
import functools

import jax
import jax.numpy as jnp
from jax import lax
from jax.experimental import pallas as pl
from jax.experimental.pallas import tpu as pltpu
from jax.experimental.pallas import tpu_sc as plsc

_SCALE = 8.0

_RING = 4


def _scale_pad_body(w_ref, o_ref):
    w = w_ref[...]
    o_ref[...] = jnp.concatenate([w * _SCALE, jnp.zeros_like(w)], axis=1)


def _scale_pad_table(w):
    v, d = w.shape
    blk = 10000
    assert v % blk == 0 and blk % 8 == 0
    return pl.pallas_call(
        _scale_pad_body,
        out_shape=jax.ShapeDtypeStruct((v, 2 * d), w.dtype),
        grid=(v // blk,),
        in_specs=[pl.BlockSpec((blk, d), lambda i: (i, 0))],
        out_specs=pl.BlockSpec((blk, 2 * d), lambda i: (i, 0)),
    )(w)


@functools.partial(jax.jit, static_argnums=(2, 3, 4))
def _sc_gather_t(idx_t, table, b, l, d):
    info = plsc.get_sparse_core_info()
    nl = info.num_lanes
    nw = info.num_cores * info.num_subcores
    pbw = b // nw
    ng = l // _RING
    assert pbw * nw == b and ng * _RING == l and pbw % nl == 0

    mesh = plsc.VectorSubcoreMesh(core_axis_name="c", subcore_axis_name="s")

    @functools.partial(
        pl.kernel,
        mesh=mesh,
        compiler_params=pltpu.CompilerParams(needs_layout_passes=False),
        out_type=jax.ShapeDtypeStruct((l, d, b), jnp.float32),
        scratch_types=(
            [pltpu.VMEM((l, pbw), jnp.int32)]
            + [pltpu.VMEM((pbw, 2 * d), jnp.float32) for _ in range(_RING)]
            + [pltpu.VMEM((d, pbw), jnp.float32)]
            + [pltpu.SemaphoreType.DMA for _ in range(_RING)]
        ),
    )
    def k(idxt_hbm, table_hbm, out_hbm, idx_v, *rest):
        gbufs = rest[:_RING]
        sbuf = rest[_RING]
        sems = rest[_RING + 1:]
        wid = lax.axis_index("s") * info.num_cores + lax.axis_index("c")
        base = wid * pbw
        pltpu.sync_copy(idxt_hbm.at[:, pl.ds(base, pbw)], idx_v)

        def fire(i, r):
            pltpu.async_copy(table_hbm.at[idx_v.at[i]], gbufs[r], sems[r])

        def drain(i, r):
            pltpu.make_async_copy(
                table_hbm.at[idx_v.at[i]], gbufs[r], sems[r]).wait()

        def transpose(r):
            iota = lax.iota(jnp.int32, nl)
            perm = [(iota + k) & (nl - 1) for k in range(nl)]

            @plsc.parallel_loop(0, pbw // nl, unroll=4)
            def _(jb):
                rows = jb * nl + iota
                for e0 in range(0, d, nl):
                    for k in range(nl):
                        cols = perm[k] + e0
                        vals = plsc.load_gather(gbufs[r], [rows, cols])
                        plsc.store_scatter(sbuf, [cols, rows], vals)

        for r in range(_RING):
            fire(r, r)

        def body(g, carry):
            for r in range(_RING):
                i = g * _RING + r
                drain(i, r)
                transpose(r)
                pltpu.sync_copy(sbuf, out_hbm.at[i, :, pl.ds(base, pbw)])
                ni = i + _RING

                @pl.when(ni < l)
                def _():
                    fire(ni, r)
            return carry

        lax.fori_loop(0, ng, body, None)

    return k(idx_t, table)


def kernel(tokens, word_vectors):
    b, l = tokens.shape
    v, d = word_vectors.shape
    scaled = _scale_pad_table(word_vectors)
    out3 = _sc_gather_t(tokens.T, scaled, b, l, d)
    return jnp.transpose(out3, (2, 0, 1))


# speedup vs baseline: 2.2058x; 1.0216x over previous
import functools

import jax
import jax.numpy as jnp
from jax import lax
from jax.experimental import pallas as pl
from jax.experimental.pallas import tpu as pltpu
from jax.experimental.pallas import tpu_sc as plsc

_SCALE = 8.0

_RING = 5


def _scale_pad_body(w_ref, o_ref):
    w = w_ref[...]
    o_ref[...] = jnp.concatenate([w * _SCALE, jnp.zeros_like(w)], axis=1)


def _scale_pad_table(w):
    v, d = w.shape
    blk = 10000
    assert v % blk == 0 and blk % 8 == 0
    return pl.pallas_call(
        _scale_pad_body,
        out_shape=jax.ShapeDtypeStruct((v, 2 * d), w.dtype),
        grid=(v // blk,),
        in_specs=[pl.BlockSpec((blk, d), lambda i: (i, 0))],
        out_specs=pl.BlockSpec((blk, 2 * d), lambda i: (i, 0)),
    )(w)


@functools.partial(jax.jit, static_argnums=(2, 3, 4))
def _sc_gather_t(idx_t, table, b, l, d):
    info = plsc.get_sparse_core_info()
    nl = info.num_lanes
    nw = info.num_cores * info.num_subcores
    pbw = b // nw
    ng = l // _RING
    assert pbw * nw == b and ng * _RING == l and pbw % nl == 0

    mesh = plsc.VectorSubcoreMesh(core_axis_name="c", subcore_axis_name="s")

    @functools.partial(
        pl.kernel,
        mesh=mesh,
        compiler_params=pltpu.CompilerParams(needs_layout_passes=False),
        out_type=jax.ShapeDtypeStruct((l, d, b), jnp.float32),
        scratch_types=(
            [pltpu.VMEM((l, pbw), jnp.int32)]
            + [pltpu.VMEM((pbw, 2 * d), jnp.float32) for _ in range(_RING)]
            + [pltpu.VMEM((d, pbw), jnp.float32)]
            + [pltpu.SemaphoreType.DMA for _ in range(_RING)]
        ),
    )
    def k(idxt_hbm, table_hbm, out_hbm, idx_v, *rest):
        gbufs = rest[:_RING]
        sbuf = rest[_RING]
        sems = rest[_RING + 1:]
        wid = lax.axis_index("s") * info.num_cores + lax.axis_index("c")
        base = wid * pbw
        pltpu.sync_copy(idxt_hbm.at[:, pl.ds(base, pbw)], idx_v)

        def fire(i, r):
            pltpu.async_copy(table_hbm.at[idx_v.at[i]], gbufs[r], sems[r])

        def drain(i, r):
            pltpu.make_async_copy(
                table_hbm.at[idx_v.at[i]], gbufs[r], sems[r]).wait()

        def transpose(r):
            iota = lax.iota(jnp.int32, nl)
            perm = [(iota + k) & (nl - 1) for k in range(nl)]

            @plsc.parallel_loop(0, pbw // nl, unroll=4)
            def _(jb):
                rows = jb * nl + iota
                for e0 in range(0, d, nl):
                    for k in range(nl):
                        cols = perm[k] + e0
                        vals = plsc.load_gather(gbufs[r], [rows, cols])
                        plsc.store_scatter(sbuf, [cols, rows], vals)

        for r in range(_RING):
            fire(r, r)

        def body(g, carry):
            for r in range(_RING):
                i = g * _RING + r
                drain(i, r)
                transpose(r)
                pltpu.sync_copy(sbuf, out_hbm.at[i, :, pl.ds(base, pbw)])
                ni = i + _RING

                @pl.when(ni < l)
                def _():
                    fire(ni, r)
            return carry

        lax.fori_loop(0, ng, body, None)

    return k(idx_t, table)


def kernel(tokens, word_vectors):
    b, l = tokens.shape
    v, d = word_vectors.shape
    scaled = _scale_pad_table(word_vectors)
    out3 = _sc_gather_t(tokens.T, scaled, b, l, d)
    return jnp.transpose(out3, (2, 0, 1))


# speedup vs baseline: 2.4706x; 1.1201x over previous
import functools

import jax
import jax.numpy as jnp
from jax import lax
from jax.experimental import pallas as pl
from jax.experimental.pallas import tpu as pltpu
from jax.experimental.pallas import tpu_sc as plsc

_SCALE = 8.0

_RING = 4


def _scale_pad_body(w_ref, o_ref):
    w = w_ref[...]
    o_ref[...] = jnp.concatenate([w * _SCALE, jnp.zeros_like(w)], axis=1)


def _scale_pad_table(w):
    v, d = w.shape
    blk = 10000
    assert v % blk == 0 and blk % 8 == 0
    return pl.pallas_call(
        _scale_pad_body,
        out_shape=jax.ShapeDtypeStruct((v, 2 * d), w.dtype),
        grid=(v // blk,),
        in_specs=[pl.BlockSpec((blk, d), lambda i: (i, 0))],
        out_specs=pl.BlockSpec((blk, 2 * d), lambda i: (i, 0)),
    )(w)


@functools.partial(jax.jit, static_argnums=(2, 3, 4))
def _sc_gather_t(idx_t, table, b, l, d):
    info = plsc.get_sparse_core_info()
    nl = info.num_lanes
    nw = info.num_cores * info.num_subcores
    pbw = b // nw
    ng = l // _RING
    assert pbw * nw == b and ng * _RING == l and pbw % nl == 0

    mesh = plsc.VectorSubcoreMesh(core_axis_name="c", subcore_axis_name="s")

    @functools.partial(
        pl.kernel,
        mesh=mesh,
        compiler_params=pltpu.CompilerParams(needs_layout_passes=False),
        out_type=jax.ShapeDtypeStruct((l, d, b), jnp.float32),
        scratch_types=(
            [pltpu.VMEM((l, pbw), jnp.int32)]
            + [pltpu.VMEM((pbw, 2 * d), jnp.float32) for _ in range(_RING)]
            + [pltpu.VMEM((d, pbw), jnp.float32) for _ in range(2)]
            + [pltpu.SemaphoreType.DMA for _ in range(_RING + 2)]
        ),
    )
    def k(idxt_hbm, table_hbm, out_hbm, idx_v, *rest):
        gbufs = rest[:_RING]
        sbufs = rest[_RING:_RING + 2]
        sems = rest[_RING + 2:2 * _RING + 2]
        osems = rest[2 * _RING + 2:]
        wid = lax.axis_index("s") * info.num_cores + lax.axis_index("c")
        base = wid * pbw
        pltpu.sync_copy(idxt_hbm.at[:, pl.ds(base, pbw)], idx_v)

        def fire(i, r):
            pltpu.async_copy(table_hbm.at[idx_v.at[i]], gbufs[r], sems[r])

        def drain(i, r):
            pltpu.make_async_copy(
                table_hbm.at[idx_v.at[i]], gbufs[r], sems[r]).wait()

        def transpose(r, s):
            iota = lax.iota(jnp.int32, nl)
            perm = [(iota + k) & (nl - 1) for k in range(nl)]

            @plsc.parallel_loop(0, pbw // nl, unroll=4)
            def _(jb):
                rows = jb * nl + iota
                for e0 in range(0, d, nl):
                    for k in range(nl):
                        cols = perm[k] + e0
                        vals = plsc.load_gather(gbufs[r], [rows, cols])
                        plsc.store_scatter(sbufs[s], [cols, rows], vals)

        for r in range(_RING):
            fire(r, r)

        def body(g, carry):
            for r in range(_RING):
                s = r % 2
                i = g * _RING + r
                drain(i, r)

                @pl.when(i >= 2)
                def _():
                    pltpu.make_async_copy(
                        sbufs[s], out_hbm.at[i - 2, :, pl.ds(base, pbw)],
                        osems[s]).wait()

                transpose(r, s)
                pltpu.async_copy(
                    sbufs[s], out_hbm.at[i, :, pl.ds(base, pbw)], osems[s])
                ni = i + _RING

                @pl.when(ni < l)
                def _():
                    fire(ni, r)
            return carry

        lax.fori_loop(0, ng, body, None)
        for s in range(2):
            pltpu.make_async_copy(
                sbufs[s], out_hbm.at[l - 2 + s, :, pl.ds(base, pbw)],
                osems[s]).wait()

    return k(idx_t, table)


def kernel(tokens, word_vectors):
    b, l = tokens.shape
    v, d = word_vectors.shape
    scaled = _scale_pad_table(word_vectors)
    out3 = _sc_gather_t(tokens.T, scaled, b, l, d)
    return jnp.transpose(out3, (2, 0, 1))
